# Initial kernel scaffold; baseline (speedup 1.0000x reference)
import jax
import jax.numpy as jnp
from jax.experimental import pallas as pl


def kernel(x, params):
    raise NotImplementedError("write your pallas kernel here")


# speedup vs baseline: 7.0213x; 7.0213x over previous
import jax
import jax.numpy as jnp
from jax.experimental import pallas as pl

_B, _L, _D = 16, 2048, 512
_H = 8
_DH = _D // _H
_FF = 2048
_MC = 128
_SCALE = 1.0 / float(_DH) ** 0.5

_ENC_KEYS = ('Wq', 'bq', 'Wk', 'bk', 'Wv', 'bv', 'Wo', 'bo',
             'Wf1', 'bf1', 'Wf2', 'bf2', 'ln1_g', 'ln1_b', 'ln2_g', 'ln2_b')
_CA_KEYS = ('Wq', 'bq', 'Wk', 'bk', 'Wv', 'bv', 'Wo', 'bo')


def _dot(a, b):
    return jax.lax.dot_general(a, b, (((1,), (0,)), ((), ())),
                               preferred_element_type=jnp.float32)


def _dot_bt(a, b):
    return jax.lax.dot_general(a, b, (((1,), (1,)), ((), ())),
                               preferred_element_type=jnp.float32)


def _dot_at(a, b):
    return jax.lax.dot_general(a, b, (((0,), (0,)), ((), ())),
                               preferred_element_type=jnp.float32)


def _ln(x, g, b, eps=1e-5):
    m = jnp.mean(x, axis=-1, keepdims=True)
    d = x - m
    v = jnp.mean(d * d, axis=-1, keepdims=True)
    return d / jnp.sqrt(v + eps) * g + b


def _mha(qin, kin, vin, p):
    q = _dot(qin, p['Wq']) + p['bq']
    k = _dot(kin, p['Wk']) + p['bk']
    v = _dot(vin, p['Wv']) + p['bv']
    outs = []
    for h in range(_H):
        sl = slice(h * _DH, (h + 1) * _DH)
        s = _dot_bt(q[:, sl], k[:, sl]) * _SCALE
        outs.append(_dot(jax.nn.softmax(s, axis=-1), v[:, sl]))
    o = jnp.concatenate(outs, axis=1)
    return _dot(o, p['Wo']) + p['bo']


def _enc(x, p):
    a = _mha(x, x, x, p)
    x1 = _ln(x + a, p['ln1_g'], p['ln1_b'])
    f = _dot(jnp.maximum(_dot(x1, p['Wf1']) + p['bf1'], 0.0), p['Wf2']) + p['bf2']
    return _ln(x1 + f, p['ln2_g'], p['ln2_b'])


def _cumsum_col(c):
    s = 1
    while s < _L:
        c = c + jnp.concatenate([jnp.zeros((s, 1), jnp.float32), c[:-s]], axis=0)
        s *= 2
    return c


def _seg_onehot(hard):
    seg = jnp.clip(_cumsum_col(hard) - 1.0, 0.0, float(_MC - 1)).astype(jnp.int32)
    iota = jax.lax.broadcasted_iota(jnp.int32, (_L, _MC), 1)
    return (seg == iota).astype(jnp.float32)


def _chunker_body(x_ref, w1_ref, b1_ref, w2_ref, b2_ref,
                  chunks_ref, hard_ref):
    xb = x_ref[0]
    shifted = jnp.concatenate([xb[1:], jnp.zeros((1, _D), jnp.float32)], axis=0)
    dot = jnp.sum(xb * shifted, axis=1, keepdims=True)
    nrm = jnp.maximum(jnp.sqrt(jnp.sum(xb * xb, axis=1, keepdims=True)), 1e-8)
    nrm_next = jnp.concatenate([nrm[1:], jnp.ones((1, 1), jnp.float32)], axis=0)
    bprob = 0.5 * (1.0 - dot / (nrm * nrm_next))
    binp = jnp.concatenate([xb, shifted], axis=1)
    h = jnp.maximum(_dot(binp, w1_ref[...]) + b1_ref[...], 0.0)
    learned = jax.nn.sigmoid(_dot(h, w2_ref[...]) + b2_ref[...])
    v = 0.7 * bprob + 0.3 * learned
    fb = jnp.concatenate([jnp.ones((1, 1), jnp.float32), v[:-1]], axis=0)
    hard = (fb > 0.5).astype(jnp.float32)
    m = _seg_onehot(hard)
    ssum = _dot_at(m, xb)
    cnt = _dot_at(m, jnp.ones((_L, 1), jnp.float32))
    chunks_ref[0] = ssum / jnp.maximum(cnt, 1.0)
    hard_ref[0] = hard


def _backbone_body(*refs):
    chunks_ref, hard_ref = refs[0], refs[1]
    out_ref = refs[-1]
    vals = [r[...] for r in refs[2:-1]]
    l1 = dict(zip(_ENC_KEYS, vals[0:16]))
    l2 = dict(zip(_ENC_KEYS, vals[16:32]))
    ca = dict(zip(_CA_KEYS, vals[32:40]))
    bb_g, bb_b = vals[40], vals[41]
    dc_w, dc_b, dc_lg, dc_lb = vals[42], vals[43], vals[44], vals[45]

    cb = chunks_ref[0]
    h1 = _enc(cb, l1)
    h2 = _enc(h1, l2)
    cav = _mha(h2, h1, h1, ca)
    hout = _ln(h2 + cav, bb_g, bb_b)
    proj = _dot(hout, dc_w) + dc_b

    m = _seg_onehot(hard_ref[0])
    tokens = _dot(m, proj)
    out_ref[0] = _ln(tokens, dc_lg, dc_lb)


def _cspec(a):
    return pl.BlockSpec(a.shape, lambda b, _n=a.ndim: (0,) * _n)


def _row(a):
    return a.reshape(1, a.shape[-1])


def kernel(x, params):
    bn = params['bn']
    b1 = _row(bn['b1'])
    b2 = bn['b2'].reshape(1, 1)

    chunk_ws = [bn['W1'], b1, bn['W2'], b2]
    chunks, hard = pl.pallas_call(
        _chunker_body,
        grid=(_B,),
        in_specs=[pl.BlockSpec((1, _L, _D), lambda b: (b, 0, 0))]
                 + [_cspec(w) for w in chunk_ws],
        out_specs=[pl.BlockSpec((1, _MC, _D), lambda b: (b, 0, 0)),
                   pl.BlockSpec((1, _L, 1), lambda b: (b, 0, 0))],
        out_shape=[jax.ShapeDtypeStruct((_B, _MC, _D), jnp.float32),
                   jax.ShapeDtypeStruct((_B, _L, 1), jnp.float32)],
    )(x, *chunk_ws)

    def enc_flat(p):
        return [p[k] if p[k].ndim == 2 else _row(p[k]) for k in _ENC_KEYS]

    ws = (enc_flat(params['l1']) + enc_flat(params['l2'])
          + [params['ca'][k] if params['ca'][k].ndim == 2 else _row(params['ca'][k])
             for k in _CA_KEYS]
          + [_row(params['bb_ln']['g']), _row(params['bb_ln']['b']),
             params['dc']['W'], _row(params['dc']['b']),
             _row(params['dc']['ln_g']), _row(params['dc']['ln_b'])])

    out = pl.pallas_call(
        _backbone_body,
        grid=(_B,),
        in_specs=[pl.BlockSpec((1, _MC, _D), lambda b: (b, 0, 0)),
                  pl.BlockSpec((1, _L, 1), lambda b: (b, 0, 0))]
                 + [_cspec(w) for w in ws],
        out_specs=pl.BlockSpec((1, _L, _D), lambda b: (b, 0, 0)),
        out_shape=jax.ShapeDtypeStruct((_B, _L, _D), jnp.float32),
    )(chunks, hard, *ws)
    return out


# speedup vs baseline: 9.6779x; 1.3784x over previous
import jax
import jax.numpy as jnp
from jax.experimental import pallas as pl

_B, _L, _D = 16, 2048, 512
_H = 8
_DH = _D // _H
_FF = 2048
_MC = 128
_SCALE = 1.0 / float(_DH) ** 0.5

_ENC_KEYS = ('Wq', 'bq', 'Wk', 'bk', 'Wv', 'bv', 'Wo', 'bo',
             'Wf1', 'bf1', 'Wf2', 'bf2', 'ln1_g', 'ln1_b', 'ln2_g', 'ln2_b')
_CA_KEYS = ('Wq', 'bq', 'Wk', 'bk', 'Wv', 'bv', 'Wo', 'bo')


def _dot(a, b):
    return jax.lax.dot_general(a, b, (((1,), (0,)), ((), ())),
                               preferred_element_type=jnp.float32)


def _dot_bt(a, b):
    return jax.lax.dot_general(a, b, (((1,), (1,)), ((), ())),
                               preferred_element_type=jnp.float32)


def _dot_at(a, b):
    return jax.lax.dot_general(a, b, (((0,), (0,)), ((), ())),
                               preferred_element_type=jnp.float32)


def _ln(x, g, b, eps=1e-5):
    m = jnp.mean(x, axis=-1, keepdims=True)
    d = x - m
    v = jnp.mean(d * d, axis=-1, keepdims=True)
    return d / jnp.sqrt(v + eps) * g + b


def _mha(qin, kin, vin, p, nb):
    q = _dot(qin, p['Wq']) + p['bq']
    k = _dot(kin, p['Wk']) + p['bk']
    v = _dot(vin, p['Wv']) + p['bv']
    outs = []
    for h in range(_H):
        sl = slice(h * _DH, (h + 1) * _DH)
        qh = q[:, sl].reshape(nb, _MC, _DH)
        kh = k[:, sl].reshape(nb, _MC, _DH)
        vh = v[:, sl].reshape(nb, _MC, _DH)
        s = jax.lax.dot_general(qh, kh, (((2,), (2,)), ((0,), (0,))),
                                preferred_element_type=jnp.float32) * _SCALE
        a = jax.nn.softmax(s, axis=-1)
        oh = jax.lax.dot_general(a, vh, (((2,), (1,)), ((0,), (0,))),
                                 preferred_element_type=jnp.float32)
        outs.append(oh.reshape(nb * _MC, _DH))
    o = jnp.concatenate(outs, axis=1)
    return _dot(o, p['Wo']) + p['bo']


def _enc(x, p, nb):
    a = _mha(x, x, x, p, nb)
    x1 = _ln(x + a, p['ln1_g'], p['ln1_b'])
    f = _dot(jnp.maximum(_dot(x1, p['Wf1']) + p['bf1'], 0.0), p['Wf2']) + p['bf2']
    return _ln(x1 + f, p['ln2_g'], p['ln2_b'])


def _cumsum_col(c):
    s = 1
    while s < _L:
        c = c + jnp.concatenate([jnp.zeros((s, 1), jnp.float32), c[:-s]], axis=0)
        s *= 2
    return c


def _seg_onehot(hard):
    seg = jnp.clip(_cumsum_col(hard) - 1.0, 0.0, float(_MC - 1)).astype(jnp.int32)
    iota = jax.lax.broadcasted_iota(jnp.int32, (_L, _MC), 1)
    return (seg == iota).astype(jnp.float32)


def _chunker_body(x_ref, w1_ref, b1_ref, w2_ref, b2_ref,
                  chunks_ref, hard_ref):
    xb = x_ref[0]
    shifted = jnp.concatenate([xb[1:], jnp.zeros((1, _D), jnp.float32)], axis=0)
    dot = jnp.sum(xb * shifted, axis=1, keepdims=True)
    nrm = jnp.maximum(jnp.sqrt(jnp.sum(xb * xb, axis=1, keepdims=True)), 1e-8)
    nrm_next = jnp.concatenate([nrm[1:], jnp.ones((1, 1), jnp.float32)], axis=0)
    bprob = 0.5 * (1.0 - dot / (nrm * nrm_next))
    binp = jnp.concatenate([xb, shifted], axis=1)
    h = jnp.maximum(_dot(binp, w1_ref[...]) + b1_ref[...], 0.0)
    learned = jax.nn.sigmoid(_dot(h, w2_ref[...]) + b2_ref[...])
    v = 0.7 * bprob + 0.3 * learned
    fb = jnp.concatenate([jnp.ones((1, 1), jnp.float32), v[:-1]], axis=0)
    hard = (fb > 0.5).astype(jnp.float32)
    m = _seg_onehot(hard)
    ssum = _dot_at(m, xb)
    cnt = _dot_at(m, jnp.ones((_L, 1), jnp.float32))
    chunks_ref[0] = ssum / jnp.maximum(cnt, 1.0)
    hard_ref[0] = hard


_GB = 8


def _backbone_body(*refs):
    chunks_ref = refs[0]
    proj_ref = refs[-1]
    vals = [r[...] for r in refs[1:-1]]
    l1 = dict(zip(_ENC_KEYS, vals[0:16]))
    l2 = dict(zip(_ENC_KEYS, vals[16:32]))
    ca = dict(zip(_CA_KEYS, vals[32:40]))
    bb_g, bb_b = vals[40], vals[41]
    dc_w, dc_b = vals[42], vals[43]

    cb = chunks_ref[...].reshape(_GB * _MC, _D)
    h1 = _enc(cb, l1, _GB)
    h2 = _enc(h1, l2, _GB)
    cav = _mha(h2, h1, h1, ca, _GB)
    hout = _ln(h2 + cav, bb_g, bb_b)
    proj_ref[...] = (_dot(hout, dc_w) + dc_b).reshape(_GB, _MC, _D)


def _dechunk_body(hard_ref, proj_ref, dc_lg_ref, dc_lb_ref, out_ref):
    m = _seg_onehot(hard_ref[0])
    tokens = _dot(m, proj_ref[0])
    out_ref[0] = _ln(tokens, dc_lg_ref[...], dc_lb_ref[...])


def _cspec(a):
    return pl.BlockSpec(a.shape, lambda b, _n=a.ndim: (0,) * _n)


def _row(a):
    return a.reshape(1, a.shape[-1])


def kernel(x, params):
    bn = params['bn']
    b1 = _row(bn['b1'])
    b2 = bn['b2'].reshape(1, 1)

    chunk_ws = [bn['W1'], b1, bn['W2'], b2]
    chunks, hard = pl.pallas_call(
        _chunker_body,
        grid=(_B,),
        in_specs=[pl.BlockSpec((1, _L, _D), lambda b: (b, 0, 0))]
                 + [_cspec(w) for w in chunk_ws],
        out_specs=[pl.BlockSpec((1, _MC, _D), lambda b: (b, 0, 0)),
                   pl.BlockSpec((1, _L, 1), lambda b: (b, 0, 0))],
        out_shape=[jax.ShapeDtypeStruct((_B, _MC, _D), jnp.float32),
                   jax.ShapeDtypeStruct((_B, _L, 1), jnp.float32)],
    )(x, *chunk_ws)

    def enc_flat(p):
        return [p[k] if p[k].ndim == 2 else _row(p[k]) for k in _ENC_KEYS]

    ws = (enc_flat(params['l1']) + enc_flat(params['l2'])
          + [params['ca'][k] if params['ca'][k].ndim == 2 else _row(params['ca'][k])
             for k in _CA_KEYS]
          + [_row(params['bb_ln']['g']), _row(params['bb_ln']['b']),
             params['dc']['W'], _row(params['dc']['b'])])

    proj = pl.pallas_call(
        _backbone_body,
        grid=(_B // _GB,),
        in_specs=[pl.BlockSpec((_GB, _MC, _D), lambda b: (b, 0, 0))]
                 + [_cspec(w) for w in ws],
        out_specs=pl.BlockSpec((_GB, _MC, _D), lambda b: (b, 0, 0)),
        out_shape=jax.ShapeDtypeStruct((_B, _MC, _D), jnp.float32),
    )(chunks, *ws)

    dc_lg = _row(params['dc']['ln_g'])
    dc_lb = _row(params['dc']['ln_b'])
    out = pl.pallas_call(
        _dechunk_body,
        grid=(_B,),
        in_specs=[pl.BlockSpec((1, _L, 1), lambda b: (b, 0, 0)),
                  pl.BlockSpec((1, _MC, _D), lambda b: (b, 0, 0)),
                  _cspec(dc_lg), _cspec(dc_lb)],
        out_specs=pl.BlockSpec((1, _L, _D), lambda b: (b, 0, 0)),
        out_shape=jax.ShapeDtypeStruct((_B, _L, _D), jnp.float32),
    )(hard, proj, dc_lg, dc_lb)
    return out


# speedup vs baseline: 10.2804x; 1.0623x over previous
import jax
import jax.numpy as jnp
from jax.experimental import pallas as pl

_B, _L, _D = 16, 2048, 512
_H = 8
_DH = _D // _H
_FF = 2048
_MC = 128
_SCALE = 1.0 / float(_DH) ** 0.5

_ENC_KEYS = ('Wq', 'bq', 'Wk', 'bk', 'Wv', 'bv', 'Wo', 'bo',
             'Wf1', 'bf1', 'Wf2', 'bf2', 'ln1_g', 'ln1_b', 'ln2_g', 'ln2_b')
_CA_KEYS = ('Wq', 'bq', 'Wk', 'bk', 'Wv', 'bv', 'Wo', 'bo')


def _dot(a, b):
    return jax.lax.dot_general(a, b, (((1,), (0,)), ((), ())),
                               preferred_element_type=jnp.float32)


def _dot_bt(a, b):
    return jax.lax.dot_general(a, b, (((1,), (1,)), ((), ())),
                               preferred_element_type=jnp.float32)


def _dot_at(a, b):
    return jax.lax.dot_general(a, b, (((0,), (0,)), ((), ())),
                               preferred_element_type=jnp.float32)


def _ln(x, g, b, eps=1e-5):
    m = jnp.mean(x, axis=-1, keepdims=True)
    d = x - m
    v = jnp.mean(d * d, axis=-1, keepdims=True)
    return d / jnp.sqrt(v + eps) * g + b


def _mha(qin, kin, vin, p, nb):
    q = _dot(qin, p['Wq']) + p['bq']
    k = _dot(kin, p['Wk']) + p['bk']
    v = _dot(vin, p['Wv']) + p['bv']
    outs = []
    for h in range(_H):
        sl = slice(h * _DH, (h + 1) * _DH)
        qh = q[:, sl].reshape(nb, _MC, _DH)
        kh = k[:, sl].reshape(nb, _MC, _DH)
        vh = v[:, sl].reshape(nb, _MC, _DH)
        s = jax.lax.dot_general(qh, kh, (((2,), (2,)), ((0,), (0,))),
                                preferred_element_type=jnp.float32) * _SCALE
        a = jax.nn.softmax(s, axis=-1)
        oh = jax.lax.dot_general(a, vh, (((2,), (1,)), ((0,), (0,))),
                                 preferred_element_type=jnp.float32)
        outs.append(oh.reshape(nb * _MC, _DH))
    o = jnp.concatenate(outs, axis=1)
    return _dot(o, p['Wo']) + p['bo']


def _enc(x, p, nb):
    a = _mha(x, x, x, p, nb)
    x1 = _ln(x + a, p['ln1_g'], p['ln1_b'])
    f = _dot(jnp.maximum(_dot(x1, p['Wf1']) + p['bf1'], 0.0), p['Wf2']) + p['bf2']
    return _ln(x1 + f, p['ln2_g'], p['ln2_b'])


def _cumsum_col(c):
    s = 1
    while s < _L:
        c = c + jnp.concatenate([jnp.zeros((s, 1), jnp.float32), c[:-s]], axis=0)
        s *= 2
    return c


def _seg_onehot(hard):
    seg = jnp.clip(_cumsum_col(hard) - 1.0, 0.0, float(_MC - 1)).astype(jnp.int32)
    iota = jax.lax.broadcasted_iota(jnp.int32, (_L, _MC), 1)
    return (seg == iota).astype(jnp.float32)


_GC = 2


def _chunker_body(x_ref, w1_ref, b1_ref, w2_ref, b2_ref,
                  chunks_ref, hard_ref):
    binps = []
    bprobs = []
    for g in range(_GC):
        xb = x_ref[g]
        shifted = jnp.concatenate([xb[1:], jnp.zeros((1, _D), jnp.float32)],
                                  axis=0)
        dot = jnp.sum(xb * shifted, axis=1, keepdims=True)
        nrm = jnp.maximum(jnp.sqrt(jnp.sum(xb * xb, axis=1, keepdims=True)),
                          1e-8)
        nrm_next = jnp.concatenate([nrm[1:], jnp.ones((1, 1), jnp.float32)],
                                   axis=0)
        bprobs.append(0.5 * (1.0 - dot / (nrm * nrm_next)))
        binps.append(jnp.concatenate([xb, shifted], axis=1))
    binp = jnp.concatenate(binps, axis=0)
    h = jnp.maximum(_dot(binp, w1_ref[...]) + b1_ref[...], 0.0)
    learned = jax.nn.sigmoid(_dot(h, w2_ref[...]) + b2_ref[...])
    for g in range(_GC):
        xb = x_ref[g]
        v = 0.7 * bprobs[g] + 0.3 * learned[g * _L:(g + 1) * _L]
        fb = jnp.concatenate([jnp.ones((1, 1), jnp.float32), v[:-1]], axis=0)
        hard = (fb > 0.5).astype(jnp.float32)
        m = _seg_onehot(hard)
        ssum = _dot_at(m, xb)
        cnt = _dot_at(m, jnp.ones((_L, 1), jnp.float32))
        chunks_ref[g] = ssum / jnp.maximum(cnt, 1.0)
        hard_ref[g] = hard


_GB = 8


def _backbone_body(*refs):
    chunks_ref = refs[0]
    proj_ref = refs[-1]
    vals = [r[...] for r in refs[1:-1]]
    l1 = dict(zip(_ENC_KEYS, vals[0:16]))
    l2 = dict(zip(_ENC_KEYS, vals[16:32]))
    ca = dict(zip(_CA_KEYS, vals[32:40]))
    bb_g, bb_b = vals[40], vals[41]
    dc_w, dc_b, dc_lg, dc_lb = vals[42], vals[43], vals[44], vals[45]

    cb = chunks_ref[...].reshape(_GB * _MC, _D)
    h1 = _enc(cb, l1, _GB)
    h2 = _enc(h1, l2, _GB)
    cav = _mha(h2, h1, h1, ca, _GB)
    hout = _ln(h2 + cav, bb_g, bb_b)
    proj = _ln(_dot(hout, dc_w) + dc_b, dc_lg, dc_lb)
    proj_ref[...] = proj.reshape(_GB, _MC, _D)


def _dechunk_body(hard_ref, proj_ref, out_ref):
    m = _seg_onehot(hard_ref[0])
    out_ref[0] = _dot(m, proj_ref[0])


def _cspec(a):
    return pl.BlockSpec(a.shape, lambda b, _n=a.ndim: (0,) * _n)


def _row(a):
    return a.reshape(1, a.shape[-1])


def kernel(x, params):
    bn = params['bn']
    b1 = _row(bn['b1'])
    b2 = bn['b2'].reshape(1, 1)

    chunk_ws = [bn['W1'], b1, bn['W2'], b2]
    chunks, hard = pl.pallas_call(
        _chunker_body,
        grid=(_B // _GC,),
        in_specs=[pl.BlockSpec((_GC, _L, _D), lambda b: (b, 0, 0))]
                 + [_cspec(w) for w in chunk_ws],
        out_specs=[pl.BlockSpec((_GC, _MC, _D), lambda b: (b, 0, 0)),
                   pl.BlockSpec((_GC, _L, 1), lambda b: (b, 0, 0))],
        out_shape=[jax.ShapeDtypeStruct((_B, _MC, _D), jnp.float32),
                   jax.ShapeDtypeStruct((_B, _L, 1), jnp.float32)],
    )(x, *chunk_ws)

    def enc_flat(p):
        return [p[k] if p[k].ndim == 2 else _row(p[k]) for k in _ENC_KEYS]

    ws = (enc_flat(params['l1']) + enc_flat(params['l2'])
          + [params['ca'][k] if params['ca'][k].ndim == 2 else _row(params['ca'][k])
             for k in _CA_KEYS]
          + [_row(params['bb_ln']['g']), _row(params['bb_ln']['b']),
             params['dc']['W'], _row(params['dc']['b']),
             _row(params['dc']['ln_g']), _row(params['dc']['ln_b'])])

    proj = pl.pallas_call(
        _backbone_body,
        grid=(_B // _GB,),
        in_specs=[pl.BlockSpec((_GB, _MC, _D), lambda b: (b, 0, 0))]
                 + [_cspec(w) for w in ws],
        out_specs=pl.BlockSpec((_GB, _MC, _D), lambda b: (b, 0, 0)),
        out_shape=jax.ShapeDtypeStruct((_B, _MC, _D), jnp.float32),
    )(chunks, *ws)

    out = pl.pallas_call(
        _dechunk_body,
        grid=(_B,),
        in_specs=[pl.BlockSpec((1, _L, 1), lambda b: (b, 0, 0)),
                  pl.BlockSpec((1, _MC, _D), lambda b: (b, 0, 0))],
        out_specs=pl.BlockSpec((1, _L, _D), lambda b: (b, 0, 0)),
        out_shape=jax.ShapeDtypeStruct((_B, _L, _D), jnp.float32),
    )(hard, proj)
    return out
